# baseline (speedup 1.0000x reference)
import functools

import jax
import jax.numpy as jnp
import numpy as np
from jax.experimental import pallas as pl
from jax.experimental.pallas import tpu as pltpu

try:
    from jax import shard_map as _shard_map_fn

    def _shard_map(f, mesh, in_specs, out_specs):
        return _shard_map_fn(f, mesh=mesh, in_specs=in_specs,
                             out_specs=out_specs, check_vma=False)
except ImportError:
    from jax.experimental.shard_map import shard_map as _legacy_shard_map

    def _shard_map(f, mesh, in_specs, out_specs):
        return _legacy_shard_map(f, mesh=mesh, in_specs=in_specs,
                                 out_specs=out_specs, check_rep=False)


def _dot_t(a, b):
    return jax.lax.dot_general(a, b, (((1,), (1,)), ((), ())),
                               preferred_element_type=jnp.float32)


def _mpnn_body(xf_ref, xl_ref, ea_ref, wpre_ref, bpre_ref, w1_ref, b1_ref,
               w2_ref, b2_ref, wpost_ref, bpost_ref, out_ref, acc_ref,
               *, ti, tj, nj, n, h):
    i = pl.program_id(0)
    j = pl.program_id(1)

    eb = ea_ref[...].reshape(ti * tj, h).astype(jnp.bfloat16)
    w1 = w1_ref[...].astype(jnp.bfloat16)
    r = jax.lax.dot_general(eb, w1, (((1,), (1,)), ((), ())),
                            preferred_element_type=jnp.float32)
    r = jnp.maximum(r + b1_ref[...], 0.0)
    rsum = r.reshape(ti, tj, h).sum(axis=1)

    @pl.when(j == 0)
    def _():
        acc_ref[...] = rsum

    @pl.when(j > 0)
    def _():
        acc_ref[...] += rsum

    @pl.when(j == nj - 1)
    def _():
        xbar = jnp.mean(xf_ref[...], axis=0, keepdims=True)
        xi = xl_ref[pl.ds(i * ti, ti), :]
        wpre = wpre_ref[...]
        a = wpre[:, :h]
        b = wpre[:, h:]
        pre = _dot_t(xi, a) + _dot_t(xbar, b) + bpre_ref[...]
        rbar = acc_ref[...] * (1.0 / n)
        e = _dot_t(rbar, w2_ref[...]) + b2_ref[...]
        agg = pre + e
        out_ref[...] = _dot_t(agg, wpost_ref[...]) + bpost_ref[...]


def _mpnn_shard(x_full, x_loc, ea, wpre, bpre, w1, b1, w2, b2, wpost, bpost,
                *, n, h, ti, tj):
    nl = x_loc.shape[0]
    ni, nj = nl // ti, n // tj
    body = functools.partial(_mpnn_body, ti=ti, tj=tj, nj=nj, n=n, h=h)
    full = lambda shape: pl.BlockSpec(shape, lambda i, j: (0,) * len(shape))

    return pl.pallas_call(
        body,
        grid=(ni, nj),
        in_specs=[
            full((n, h)),
            full((nl, h)),
            pl.BlockSpec((ti, tj, h), lambda i, j: (i, j, 0)),
            full((h, 2 * h)),
            full((1, h)),
            full((h, h)),
            full((1, h)),
            full((h, h)),
            full((1, h)),
            full((h, h)),
            full((1, h)),
        ],
        out_specs=pl.BlockSpec((ti, h), lambda i, j: (i, 0)),
        out_shape=jax.ShapeDtypeStruct((nl, h), jnp.float32),
        scratch_shapes=[pltpu.VMEM((ti, h), jnp.float32)],
        compiler_params=pltpu.CompilerParams(
            dimension_semantics=("parallel", "arbitrary"),
        ),
    )(x_full, x_loc, ea, wpre, bpre, w1, b1, w2, b2, wpost, bpost)


def kernel(x, adj, edge_attr, W_pre, b_pre, W1, b1, W2, b2, W_post, b_post):
    del adj
    n, h = x.shape
    ti, tj = 64, 256

    devs = jax.devices()
    nd = 2 if len(devs) >= 2 and n % 2 == 0 else 1
    mesh = jax.sharding.Mesh(np.array(devs[:nd]), ("d",))
    P = jax.sharding.PartitionSpec

    fn = functools.partial(_mpnn_shard, n=n, h=h, ti=ti, tj=tj)
    sharded = _shard_map(
        fn,
        mesh,
        in_specs=(P(), P("d", None), P("d", None, None),
                  P(), P(), P(), P(), P(), P(), P(), P()),
        out_specs=P("d", None),
    )
    return sharded(x, x, edge_attr, W_pre, b_pre.reshape(1, h),
                   W1, b1.reshape(1, h), W2, b2.reshape(1, h),
                   W_post, b_post.reshape(1, h))


# speedup vs baseline: 15.2683x; 15.2683x over previous
import functools

import jax
import jax.numpy as jnp
from jax.experimental import pallas as pl
from jax.experimental.pallas import tpu as pltpu


def _dot_t(a, b):
    return jax.lax.dot_general(a, b, (((1,), (1,)), ((), ())),
                               preferred_element_type=jnp.float32)


def _body(x_ref, wpre_ref, bpre_ref, w1_ref, b1_ref, w2_ref, b2_ref,
          wpost_ref, bpost_ref, ea_hbm, out_ref, buf_ref, sem,
          *, ti, n, h, nb):
    k = pl.program_id(0)
    nsteps = pl.num_programs(0)

    def copy(step, slot):
        return pltpu.make_async_copy(
            ea_hbm.at[pl.ds(step * ti, ti)], buf_ref.at[slot], sem.at[slot])

    @pl.when(k == 0)
    def _():
        for s in range(nb - 1):
            copy(s, s).start()

    @pl.when(k + nb - 1 < nsteps)
    def _():
        s = k + nb - 1
        copy(s, jax.lax.rem(s, nb)).start()

    slot = jax.lax.rem(k, nb)
    copy(k, slot).wait()

    eb = buf_ref[slot].reshape(ti * n, h).astype(jnp.bfloat16)
    w1 = w1_ref[...].astype(jnp.bfloat16)
    r = jax.lax.dot_general(eb, w1, (((1,), (1,)), ((), ())),
                            preferred_element_type=jnp.float32)
    r = jnp.maximum(r + b1_ref[...], 0.0)
    rbar = r.reshape(ti, n, h).sum(axis=1) * (1.0 / n)

    xbar = jnp.mean(x_ref[...], axis=0, keepdims=True)
    xi = x_ref[pl.ds(k * ti, ti), :]
    wpre = wpre_ref[...]
    pre = _dot_t(xi, wpre[:, :h]) + _dot_t(xbar, wpre[:, h:]) + bpre_ref[...]
    e = _dot_t(rbar, w2_ref[...]) + b2_ref[...]
    out_ref[...] = _dot_t(pre + e, wpost_ref[...]) + bpost_ref[...]


def kernel(x, adj, edge_attr, W_pre, b_pre, W1, b1, W2, b2, W_post, b_post):
    del adj
    n, h = x.shape
    ti, nb = 32, 4
    nsteps = n // ti

    body = functools.partial(_body, ti=ti, n=n, h=h, nb=nb)
    full = lambda shape: pl.BlockSpec(shape, lambda k: (0,) * len(shape))

    out = pl.pallas_call(
        body,
        grid=(nsteps,),
        in_specs=[
            full((n, h)),
            full((h, 2 * h)),
            full((1, h)),
            full((h, h)),
            full((1, h)),
            full((h, h)),
            full((1, h)),
            full((h, h)),
            full((1, h)),
            pl.BlockSpec(memory_space=pl.ANY),
        ],
        out_specs=pl.BlockSpec((ti, h), lambda k: (k, 0)),
        out_shape=jax.ShapeDtypeStruct((n, h), jnp.float32),
        scratch_shapes=[
            pltpu.VMEM((nb, ti, n, h), jnp.float32),
            pltpu.SemaphoreType.DMA((nb,)),
        ],
        compiler_params=pltpu.CompilerParams(
            dimension_semantics=("arbitrary",),
        ),
    )(x, W_pre, b_pre.reshape(1, h), W1, b1.reshape(1, h),
      W2, b2.reshape(1, h), W_post, b_post.reshape(1, h), edge_attr)
    return out


# speedup vs baseline: 15.7425x; 1.0311x over previous
import functools

import jax
import jax.numpy as jnp
from jax.experimental import pallas as pl
from jax.experimental.pallas import tpu as pltpu


def _dot_t(a, b):
    return jax.lax.dot_general(a, b, (((1,), (1,)), ((), ())),
                               preferred_element_type=jnp.float32)


def _mpnn_body(x_ref, ea_ref, wpre_ref, bpre_ref, w1_ref, b1_ref,
               w2_ref, b2_ref, wpost_ref, bpost_ref, out_ref, acc_ref,
               *, ti, tj, nj, n, h):
    i = pl.program_id(0)
    j = pl.program_id(1)

    eb = ea_ref[...].reshape(ti * tj, h).astype(jnp.bfloat16)
    w1 = w1_ref[...].astype(jnp.bfloat16)
    r = jax.lax.dot_general(eb, w1, (((1,), (1,)), ((), ())),
                            preferred_element_type=jnp.float32)
    r = jnp.maximum(r + b1_ref[...], 0.0)
    rsum = r.reshape(ti, tj, h).sum(axis=1)

    @pl.when(j == 0)
    def _():
        acc_ref[...] = rsum

    @pl.when(j > 0)
    def _():
        acc_ref[...] += rsum

    @pl.when(j == nj - 1)
    def _():
        xall = x_ref[...]
        xbar = jnp.mean(xall, axis=0, keepdims=True)
        xi = x_ref[pl.ds(i * ti, ti), :]
        wpre = wpre_ref[...]
        a = wpre[:, :h]
        b = wpre[:, h:]
        pre = _dot_t(xi, a) + _dot_t(xbar, b) + bpre_ref[...]
        rbar = acc_ref[...] * (1.0 / n)
        e = _dot_t(rbar, w2_ref[...]) + b2_ref[...]
        agg = pre + e
        out_ref[...] = _dot_t(agg, wpost_ref[...]) + bpost_ref[...]


def kernel(x, adj, edge_attr, W_pre, b_pre, W1, b1, W2, b2, W_post, b_post):
    del adj
    n, h = x.shape
    ti, tj = 64, 256
    ni, nj = n // ti, n // tj

    grid = (ni, nj)
    body = functools.partial(_mpnn_body, ti=ti, tj=tj, nj=nj, n=n, h=h)
    full = lambda shape: pl.BlockSpec(shape, lambda i, j: (0,) * len(shape))

    out = pl.pallas_call(
        body,
        grid=grid,
        in_specs=[
            full((n, h)),
            pl.BlockSpec((ti, tj, h), lambda i, j: (i, j, 0)),
            full((h, 2 * h)),
            full((1, h)),
            full((h, h)),
            full((1, h)),
            full((h, h)),
            full((1, h)),
            full((h, h)),
            full((1, h)),
        ],
        out_specs=pl.BlockSpec((ti, h), lambda i, j: (i, 0)),
        out_shape=jax.ShapeDtypeStruct((n, h), jnp.float32),
        scratch_shapes=[pltpu.VMEM((ti, h), jnp.float32)],
        compiler_params=pltpu.CompilerParams(
            dimension_semantics=("parallel", "arbitrary"),
        ),
    )(x, edge_attr, W_pre, b_pre.reshape(1, h), W1, b1.reshape(1, h),
      W2, b2.reshape(1, h), W_post, b_post.reshape(1, h))
    return out


# speedup vs baseline: 15.7626x; 1.0013x over previous
import functools

import jax
import jax.numpy as jnp
from jax.experimental import pallas as pl
from jax.experimental.pallas import tpu as pltpu


def _dot_t(a, b):
    return jax.lax.dot_general(a, b, (((1,), (1,)), ((), ())),
                               preferred_element_type=jnp.float32)


def _mpnn_body(x_ref, ea_ref, wpre_ref, bpre_ref, w1_ref, b1_ref,
               w2_ref, b2_ref, wpost_ref, bpost_ref, out_ref, acc_ref,
               *, ti, tj, nj, n, h):
    i = pl.program_id(0)
    j = pl.program_id(1)

    eb = ea_ref[...].reshape(ti * tj, h).astype(jnp.bfloat16)
    w1 = w1_ref[...].astype(jnp.bfloat16)
    r = jax.lax.dot_general(eb, w1, (((1,), (1,)), ((), ())),
                            preferred_element_type=jnp.float32)
    r = jnp.maximum(r + b1_ref[...], 0.0)
    rsum = r.reshape(ti, tj, h).sum(axis=1)

    @pl.when(j == 0)
    def _():
        acc_ref[...] = rsum

    @pl.when(j > 0)
    def _():
        acc_ref[...] += rsum

    @pl.when(j == nj - 1)
    def _():
        xall = x_ref[...]
        xbar = jnp.mean(xall, axis=0, keepdims=True)
        xi = x_ref[pl.ds(i * ti, ti), :]
        wpre = wpre_ref[...]
        a = wpre[:, :h]
        b = wpre[:, h:]
        pre = _dot_t(xi, a) + _dot_t(xbar, b) + bpre_ref[...]
        rbar = acc_ref[...] * (1.0 / n)
        e = _dot_t(rbar, w2_ref[...]) + b2_ref[...]
        agg = pre + e
        out_ref[...] = _dot_t(agg, wpost_ref[...]) + bpost_ref[...]


def kernel(x, adj, edge_attr, W_pre, b_pre, W1, b1, W2, b2, W_post, b_post):
    del adj
    n, h = x.shape
    ti, tj = 64, 512
    ni, nj = n // ti, n // tj

    grid = (ni, nj)
    body = functools.partial(_mpnn_body, ti=ti, tj=tj, nj=nj, n=n, h=h)
    full = lambda shape: pl.BlockSpec(shape, lambda i, j: (0,) * len(shape))

    out = pl.pallas_call(
        body,
        grid=grid,
        in_specs=[
            full((n, h)),
            pl.BlockSpec((ti, tj, h), lambda i, j: (i, j, 0)),
            full((h, 2 * h)),
            full((1, h)),
            full((h, h)),
            full((1, h)),
            full((h, h)),
            full((1, h)),
            full((h, h)),
            full((1, h)),
        ],
        out_specs=pl.BlockSpec((ti, h), lambda i, j: (i, 0)),
        out_shape=jax.ShapeDtypeStruct((n, h), jnp.float32),
        scratch_shapes=[pltpu.VMEM((ti, h), jnp.float32)],
        compiler_params=pltpu.CompilerParams(
            dimension_semantics=("parallel", "arbitrary"),
        ),
    )(x, edge_attr, W_pre, b_pre.reshape(1, h), W1, b1.reshape(1, h),
      W2, b2.reshape(1, h), W_post, b_post.reshape(1, h))
    return out


# speedup vs baseline: 15.8245x; 1.0039x over previous
import functools

import jax
import jax.numpy as jnp
from jax.experimental import pallas as pl
from jax.experimental.pallas import tpu as pltpu


def _dot_t(a, b):
    return jax.lax.dot_general(a, b, (((1,), (1,)), ((), ())),
                               preferred_element_type=jnp.float32)


def _mpnn_body(x_ref, ea_ref, wpre_ref, bpre_ref, w1_ref, b1_ref,
               w2_ref, b2_ref, wpost_ref, bpost_ref, out_ref, acc_ref,
               *, ti, tj, nj, n, h):
    i = pl.program_id(0)
    j = pl.program_id(1)

    eb = ea_ref[...].reshape(ti * tj, h)
    r = jax.lax.dot_general(eb, w1_ref[...], (((1,), (1,)), ((), ())),
                            preferred_element_type=jnp.float32,
                            precision=jax.lax.Precision.DEFAULT)
    r = jnp.maximum(r + b1_ref[...], 0.0)
    rsum = r.reshape(ti, tj, h).sum(axis=1)

    @pl.when(j == 0)
    def _():
        acc_ref[...] = rsum

    @pl.when(j > 0)
    def _():
        acc_ref[...] += rsum

    @pl.when(j == nj - 1)
    def _():
        xall = x_ref[...]
        xbar = jnp.mean(xall, axis=0, keepdims=True)
        xi = x_ref[pl.ds(i * ti, ti), :]
        wpre = wpre_ref[...]
        a = wpre[:, :h]
        b = wpre[:, h:]
        pre = _dot_t(xi, a) + _dot_t(xbar, b) + bpre_ref[...]
        rbar = acc_ref[...] * (1.0 / n)
        e = _dot_t(rbar, w2_ref[...]) + b2_ref[...]
        agg = pre + e
        out_ref[...] = _dot_t(agg, wpost_ref[...]) + bpost_ref[...]


def kernel(x, adj, edge_attr, W_pre, b_pre, W1, b1, W2, b2, W_post, b_post):
    del adj
    n, h = x.shape
    ti, tj = 64, 512
    ni, nj = n // ti, n // tj

    grid = (ni, nj)
    body = functools.partial(_mpnn_body, ti=ti, tj=tj, nj=nj, n=n, h=h)
    full = lambda shape: pl.BlockSpec(shape, lambda i, j: (0,) * len(shape))

    out = pl.pallas_call(
        body,
        grid=grid,
        in_specs=[
            full((n, h)),
            pl.BlockSpec((ti, tj, h), lambda i, j: (i, j, 0)),
            full((h, 2 * h)),
            full((1, h)),
            full((h, h)),
            full((1, h)),
            full((h, h)),
            full((1, h)),
            full((h, h)),
            full((1, h)),
        ],
        out_specs=pl.BlockSpec((ti, h), lambda i, j: (i, 0)),
        out_shape=jax.ShapeDtypeStruct((n, h), jnp.float32),
        scratch_shapes=[pltpu.VMEM((ti, h), jnp.float32)],
        compiler_params=pltpu.CompilerParams(
            dimension_semantics=("parallel", "arbitrary"),
        ),
    )(x, edge_attr, W_pre, b_pre.reshape(1, h), W1, b1.reshape(1, h),
      W2, b2.reshape(1, h), W_post, b_post.reshape(1, h))
    return out
